# baseline (speedup 1.0000x reference)
import dataclasses

import jax
import jax.numpy as jnp
from jax import lax
from jax.experimental import pallas as pl
from jax.experimental.pallas import tpu as pltpu
from jax.experimental.pallas import tpu_sc as plsc

NUM_CLASSES_ = 1000
N_ROWS = 16384
N_TILES = 32
ROWS_PER_TILE = N_ROWS // N_TILES
CHUNK_ROWS = 32
CHUNK_WORDS = CHUNK_ROWS * NUM_CLASSES_
CHUNKS_PER_TILE = ROWS_PER_TILE // CHUNK_ROWS
NBUF = 2
GROUPS = CHUNK_ROWS // 16


def _one_hot_body(idx_hbm, onoff_hbm, out_hbm, idx_v, onoff_v, buf, sem0, sem1):
    c = lax.axis_index("c")
    s = lax.axis_index("s")
    wid = c * 16 + s
    row_base = wid * ROWS_PER_TILE
    flat_base = row_base * NUM_CLASSES_

    pltpu.sync_copy(idx_hbm.at[pl.ds(row_base, ROWS_PER_TILE)], idx_v)
    pltpu.sync_copy(onoff_hbm, onoff_v)
    on_vec = onoff_v[pl.ds(0, 16)]
    off_vec = onoff_v[pl.ds(16, 16)]
    row_off = lax.iota(jnp.int32, 16) * NUM_CLASSES_

    @pl.loop(0, NBUF * CHUNK_WORDS, step=128)
    def _(j):
        for d in range(8):
            buf[pl.ds(j + 16 * d, 16)] = off_vec

    sems = (sem0, sem1)

    def chunk_offsets(i, slot, g):
        cols = idx_v[pl.ds(i * CHUNK_ROWS + g * 16, 16)]
        return row_off + cols + slot * CHUNK_WORDS + g * (16 * NUM_CLASSES_)

    def copy_desc(i, slot):
        return pltpu.make_async_copy(
            buf.at[pl.ds(slot * CHUNK_WORDS, CHUNK_WORDS)],
            out_hbm.at[pl.ds(flat_base + i * CHUNK_WORDS, CHUNK_WORDS)],
            sems[slot],
        )

    def issue(i, slot):
        for g in range(GROUPS):
            plsc.store_scatter(buf, [chunk_offsets(i, slot, g)], on_vec)
        copy_desc(i, slot).start()

    def drain_restore(i, slot):
        copy_desc(i, slot).wait()
        for g in range(GROUPS):
            plsc.store_scatter(buf, [chunk_offsets(i, slot, g)], off_vec)

    for d in range(NBUF):
        issue(d, d)

    @pl.loop(NBUF, CHUNKS_PER_TILE, step=NBUF)
    def _(i):
        for d in range(NBUF):
            drain_restore(i + d - NBUF, d)
            issue(i + d, d)

    for d in range(NBUF):
        copy_desc(CHUNKS_PER_TILE - NBUF + d, d).wait()


def kernel(inputs, on_value, off_value):
    onoff = jnp.concatenate([
        jnp.broadcast_to(on_value.astype(jnp.float32), (16,)),
        jnp.broadcast_to(off_value.astype(jnp.float32), (16,)),
    ])
    mesh = plsc.VectorSubcoreMesh(
        core_axis_name="c", subcore_axis_name="s", num_cores=2, num_subcores=16
    )
    cp = pltpu.CompilerParams()
    if "needs_layout_passes" in pltpu.CompilerParams.__dataclass_fields__:
        cp = dataclasses.replace(cp, needs_layout_passes=False)
    k = pl.kernel(
        _one_hot_body,
        compiler_params=cp,
        out_type=jax.ShapeDtypeStruct((N_ROWS * NUM_CLASSES_,), jnp.float32),
        mesh=mesh,
        scratch_types=[
            pltpu.VMEM((ROWS_PER_TILE,), jnp.int32),
            pltpu.VMEM((2 * 16,), jnp.float32),
            pltpu.VMEM((NBUF * CHUNK_WORDS,), jnp.float32),
            pltpu.SemaphoreType.DMA,
            pltpu.SemaphoreType.DMA,
        ],
    )
    out_flat = k(inputs.astype(jnp.int32), onoff)
    return out_flat.reshape(N_ROWS, NUM_CLASSES_)


# speedup vs baseline: 1.6282x; 1.6282x over previous
import dataclasses

import jax
import jax.numpy as jnp
from jax import lax
from jax.experimental import pallas as pl
from jax.experimental.pallas import tpu as pltpu
from jax.experimental.pallas import tpu_sc as plsc

NUM_CLASSES_ = 1000
PAD_CLASSES = 1024
N_ROWS = 16384
N_TILES = 32
ROWS_PER_TILE = N_ROWS // N_TILES
CHUNK_ROWS = 16
CHUNKS_PER_TILE = ROWS_PER_TILE // CHUNK_ROWS
NBUF = 2


def _one_hot_body(idx_hbm, onoff_hbm, out_hbm, idx_v, onoff_v, buf, sem0, sem1):
    c = lax.axis_index("c")
    s = lax.axis_index("s")
    wid = c * 16 + s
    row_base = wid * ROWS_PER_TILE

    pltpu.sync_copy(idx_hbm.at[pl.ds(row_base, ROWS_PER_TILE)], idx_v)
    pltpu.sync_copy(onoff_hbm, onoff_v)
    on_vec = onoff_v[pl.ds(0, 16)]
    off_vec = onoff_v[pl.ds(16, 16)]
    lane_iota = lax.iota(jnp.int32, 16)

    @pl.loop(0, NBUF * CHUNK_ROWS, step=1)
    def _(r):
        @pl.loop(0, 960, step=128)
        def _(j):
            for d in range(8):
                buf[r, pl.ds(j + 16 * d, 16)] = off_vec
        for d in range(2):
            buf[r, pl.ds(960 + 16 * d, 16)] = off_vec
        buf[r, pl.ds(NUM_CLASSES_ - 16, 16)] = off_vec

    sems = (sem0, sem1)

    def chunk_cols(i):
        return idx_v[pl.ds(i * CHUNK_ROWS, 16)]

    def chunk_rows(slot):
        return lane_iota + slot * CHUNK_ROWS

    def copy_desc(i, slot):
        return pltpu.make_async_copy(
            buf.at[pl.ds(slot * CHUNK_ROWS, CHUNK_ROWS), :],
            out_hbm.at[pl.ds(row_base + i * CHUNK_ROWS, CHUNK_ROWS), :],
            sems[slot],
        )

    def issue(i, slot):
        plsc.store_scatter(buf, [chunk_rows(slot), chunk_cols(i)], on_vec)
        copy_desc(i, slot).start()

    def drain_restore(i, slot):
        copy_desc(i, slot).wait()
        plsc.store_scatter(buf, [chunk_rows(slot), chunk_cols(i)], off_vec)

    for d in range(NBUF):
        issue(d, d)

    @pl.loop(NBUF, CHUNKS_PER_TILE, step=NBUF)
    def _(i):
        for d in range(NBUF):
            drain_restore(i + d - NBUF, d)
            issue(i + d, d)

    for d in range(NBUF):
        copy_desc(CHUNKS_PER_TILE - NBUF + d, d).wait()


def kernel(inputs, on_value, off_value):
    onoff = jnp.concatenate([
        jnp.broadcast_to(on_value.astype(jnp.float32), (16,)),
        jnp.broadcast_to(off_value.astype(jnp.float32), (16,)),
    ])
    mesh = plsc.VectorSubcoreMesh(
        core_axis_name="c", subcore_axis_name="s", num_cores=2, num_subcores=16
    )
    cp = pltpu.CompilerParams(use_tc_tiling_on_sc=True)
    if "needs_layout_passes" in pltpu.CompilerParams.__dataclass_fields__:
        cp = dataclasses.replace(cp, needs_layout_passes=False)
    k = pl.kernel(
        _one_hot_body,
        out_type=jax.ShapeDtypeStruct((N_ROWS, NUM_CLASSES_), jnp.float32),
        mesh=mesh,
        compiler_params=cp,
        scratch_types=[
            pltpu.VMEM((ROWS_PER_TILE,), jnp.int32),
            pltpu.VMEM((2 * 16,), jnp.float32),
            pltpu.VMEM((NBUF * CHUNK_ROWS, NUM_CLASSES_), jnp.float32),
            pltpu.SemaphoreType.DMA,
            pltpu.SemaphoreType.DMA,
        ],
    )
    return k(inputs.astype(jnp.int32), onoff)


# speedup vs baseline: 1.6284x; 1.0001x over previous
import dataclasses

import jax
import jax.numpy as jnp
from jax import lax
from jax.experimental import pallas as pl
from jax.experimental.pallas import tpu as pltpu
from jax.experimental.pallas import tpu_sc as plsc

NUM_CLASSES_ = 1000
PAD_CLASSES = 1024
N_ROWS = 16384
N_TILES = 32
ROWS_PER_TILE = N_ROWS // N_TILES
CHUNK_ROWS = 16
CHUNKS_PER_TILE = ROWS_PER_TILE // CHUNK_ROWS
NBUF = 2


def _one_hot_body(idx_hbm, onoff_hbm, out_hbm, idx_v, onoff_v, buf, sem0, sem1):
    c = lax.axis_index("c")
    s = lax.axis_index("s")
    wid = c * 16 + s
    row_base = wid * ROWS_PER_TILE

    pltpu.sync_copy(idx_hbm.at[pl.ds(row_base, ROWS_PER_TILE)], idx_v)
    pltpu.sync_copy(onoff_hbm, onoff_v)
    on_vec = onoff_v[pl.ds(0, 16)]
    off_vec = onoff_v[pl.ds(16, 16)]
    lane_iota = lax.iota(jnp.int32, 16)

    @pl.loop(0, NBUF * CHUNK_ROWS, step=1)
    def _(r):
        @pl.loop(0, 960, step=128)
        def _(j):
            for d in range(8):
                buf[r, pl.ds(j + 16 * d, 16)] = off_vec
        for d in range(2):
            buf[r, pl.ds(960 + 16 * d, 16)] = off_vec
        buf[r, pl.ds(NUM_CLASSES_ - 16, 16)] = off_vec

    sems = (sem0, sem1)

    def chunk_cols(i):
        return idx_v[pl.ds(i * CHUNK_ROWS, 16)]

    def chunk_rows(slot):
        return lane_iota + slot * CHUNK_ROWS

    def copy_desc(i, slot):
        return pltpu.make_async_copy(
            buf.at[pl.ds(slot * CHUNK_ROWS, CHUNK_ROWS), :],
            out_hbm.at[pl.ds(row_base + i * CHUNK_ROWS, CHUNK_ROWS), :],
            sems[slot],
        )

    def issue(i, slot):
        plsc.store_scatter(buf, [chunk_rows(slot), chunk_cols(i)], on_vec)
        copy_desc(i, slot).start()

    def drain_restore(i, slot):
        copy_desc(i, slot).wait()
        plsc.store_scatter(buf, [chunk_rows(slot), chunk_cols(i)], off_vec)

    for d in range(NBUF):
        issue(d, d)

    @pl.loop(NBUF, CHUNKS_PER_TILE, step=NBUF)
    def _(i):
        for d in range(NBUF):
            drain_restore(i + d - NBUF, d)
            issue(i + d, d)

    for d in range(NBUF):
        copy_desc(CHUNKS_PER_TILE - NBUF + d, d).wait()


def kernel(inputs, on_value, off_value):
    onoff = jnp.concatenate([
        jnp.broadcast_to(on_value.astype(jnp.float32), (16,)),
        jnp.broadcast_to(off_value.astype(jnp.float32), (16,)),
    ])
    mesh = plsc.VectorSubcoreMesh(
        core_axis_name="c", subcore_axis_name="s", num_cores=2, num_subcores=16
    )
    cp = pltpu.CompilerParams(use_tc_tiling_on_sc=True, skip_device_barrier=True)
    if "needs_layout_passes" in pltpu.CompilerParams.__dataclass_fields__:
        cp = dataclasses.replace(cp, needs_layout_passes=False)
    k = pl.kernel(
        _one_hot_body,
        out_type=jax.ShapeDtypeStruct((N_ROWS, NUM_CLASSES_), jnp.float32),
        mesh=mesh,
        compiler_params=cp,
        scratch_types=[
            pltpu.VMEM((ROWS_PER_TILE,), jnp.int32),
            pltpu.VMEM((2 * 16,), jnp.float32),
            pltpu.VMEM((NBUF * CHUNK_ROWS, NUM_CLASSES_), jnp.float32),
            pltpu.SemaphoreType.DMA,
            pltpu.SemaphoreType.DMA,
        ],
    )
    return k(inputs.astype(jnp.int32), onoff)
